# baseline (speedup 1.0000x reference)
import functools

import jax
import jax.numpy as jnp
import numpy as np
from jax.experimental import pallas as pl
from jax.experimental.pallas import tpu as pltpu

D_MODEL, D_K, D_V, N_Q = 1024, 128, 128, 64
B, T = 4, 2048
TILE_T = 512
NT = T // TILE_T


def _fused_body(x_ref, q_ref, wk_ref, wv_ref, wfc_ref, out_ref, hard_ref,
                acc_ref):
    t = pl.program_id(1)
    x = x_ref[0]
    k = jax.lax.dot_general(
        x, wk_ref[...], (((1,), (1,)), ((), ())),
        preferred_element_type=jnp.float32)
    attn = jax.lax.dot_general(
        k, q_ref[...], (((1,), (1,)), ((), ())),
        preferred_element_type=jnp.float32) * (1.0 / np.sqrt(N_Q))
    m = jnp.max(attn, axis=-1, keepdims=True)
    iota = jax.lax.broadcasted_iota(jnp.int32, attn.shape, 1)
    idx = jnp.min(jnp.where(attn == m, iota, N_Q), axis=-1, keepdims=True)
    onehot = (iota == idx).astype(jnp.float32)
    hard_ref[0] = onehot
    xsum = jax.lax.dot_general(
        onehot, x, (((0,), (0,)), ((), ())),
        preferred_element_type=jnp.float32)

    @pl.when(t == 0)
    def _init():
        acc_ref[...] = xsum

    @pl.when(t > 0)
    def _accum():
        acc_ref[...] += xsum

    @pl.when(t == NT - 1)
    def _finalize():
        vslot = jax.lax.dot_general(
            acc_ref[...], wv_ref[...], (((1,), (1,)), ((), ())),
            preferred_element_type=jnp.float32)
        out_ref[0] = jax.lax.dot_general(
            vslot, wfc_ref[...], (((1,), (1,)), ((), ())),
            preferred_element_type=jnp.float32)


@functools.partial(jax.jit, static_argnames=("interpret",))
def kernel(x, q, Wk, Wv, Wfc, interpret=False):
    out, hard = pl.pallas_call(
        _fused_body,
        grid=(B, NT),
        in_specs=[
            pl.BlockSpec((1, TILE_T, D_MODEL), lambda b, t: (b, t, 0)),
            pl.BlockSpec((N_Q, D_K), lambda b, t: (0, 0)),
            pl.BlockSpec((D_K, D_MODEL), lambda b, t: (0, 0)),
            pl.BlockSpec((D_V, D_MODEL), lambda b, t: (0, 0)),
            pl.BlockSpec((D_MODEL, D_V), lambda b, t: (0, 0)),
        ],
        out_specs=[
            pl.BlockSpec((1, N_Q, D_MODEL), lambda b, t: (b, 0, 0)),
            pl.BlockSpec((1, TILE_T, N_Q), lambda b, t: (b, t, 0)),
        ],
        out_shape=[
            jax.ShapeDtypeStruct((B, N_Q, D_MODEL), jnp.float32),
            jax.ShapeDtypeStruct((B, T, N_Q), jnp.float32),
        ],
        scratch_shapes=[pltpu.VMEM((N_Q, D_MODEL), jnp.float32)],
        compiler_params=pltpu.CompilerParams(
            dimension_semantics=("parallel", "arbitrary"),
        ),
        interpret=interpret,
    )(x, q, Wk, Wv, Wfc)
    return out, hard


# speedup vs baseline: 1.3695x; 1.3695x over previous
import functools

import jax
import jax.numpy as jnp
import numpy as np
from jax.experimental import pallas as pl
from jax.experimental.pallas import tpu as pltpu

D_MODEL, D_K, D_V, N_Q = 1024, 128, 128, 64
B, T = 4, 2048
TH = T // 2


def _route_half(x, q_ref, wk_ref):
    k = jax.lax.dot_general(
        x, wk_ref[...], (((1,), (1,)), ((), ())),
        preferred_element_type=jnp.float32)
    attn = jax.lax.dot_general(
        k, q_ref[...], (((1,), (1,)), ((), ())),
        preferred_element_type=jnp.float32) * (1.0 / np.sqrt(N_Q))
    m = jnp.max(attn, axis=-1, keepdims=True)
    iota = jax.lax.broadcasted_iota(jnp.int32, attn.shape, 1)
    idx = jnp.min(jnp.where(attn == m, iota, N_Q), axis=-1, keepdims=True)
    onehot = (iota == idx).astype(jnp.float32)
    xsum = jax.lax.dot_general(
        onehot, x, (((0,), (0,)), ((), ())),
        preferred_element_type=jnp.float32)
    return onehot, xsum


def _fused_body(xlo_ref, xhi_ref, q_ref, wk_ref, wv_ref, wfc_ref,
                out_ref, hard_ref):
    onehot_lo, xsum_lo = _route_half(xlo_ref[0], q_ref, wk_ref)
    hard_ref[0, :TH] = onehot_lo
    onehot_hi, xsum_hi = _route_half(xhi_ref[0], q_ref, wk_ref)
    hard_ref[0, TH:] = onehot_hi
    xsum = xsum_lo + xsum_hi
    vslot = jax.lax.dot_general(
        xsum, wv_ref[...], (((1,), (1,)), ((), ())),
        preferred_element_type=jnp.float32)
    out_ref[0] = jax.lax.dot_general(
        vslot, wfc_ref[...], (((1,), (1,)), ((), ())),
        preferred_element_type=jnp.float32)


@functools.partial(jax.jit, static_argnames=("interpret",))
def kernel(x, q, Wk, Wv, Wfc, interpret=False):
    out, hard = pl.pallas_call(
        _fused_body,
        grid=(B,),
        in_specs=[
            pl.BlockSpec((1, TH, D_MODEL), lambda b: (b, 0, 0)),
            pl.BlockSpec((1, TH, D_MODEL), lambda b: (b, 1, 0)),
            pl.BlockSpec((N_Q, D_K), lambda b: (0, 0)),
            pl.BlockSpec((D_K, D_MODEL), lambda b: (0, 0)),
            pl.BlockSpec((D_V, D_MODEL), lambda b: (0, 0)),
            pl.BlockSpec((D_MODEL, D_V), lambda b: (0, 0)),
        ],
        out_specs=[
            pl.BlockSpec((1, N_Q, D_MODEL), lambda b: (b, 0, 0)),
            pl.BlockSpec((1, T, N_Q), lambda b: (b, 0, 0)),
        ],
        out_shape=[
            jax.ShapeDtypeStruct((B, N_Q, D_MODEL), jnp.float32),
            jax.ShapeDtypeStruct((B, T, N_Q), jnp.float32),
        ],
        compiler_params=pltpu.CompilerParams(
            dimension_semantics=("arbitrary",),
        ),
        interpret=interpret,
    )(x, x, q, Wk, Wv, Wfc)
    return out, hard
